# baseline (speedup 1.0000x reference)
import functools

import jax
import jax.numpy as jnp
from jax import lax
from jax.experimental import pallas as pl
from jax.experimental.pallas import tpu as pltpu
from jax.experimental.pallas import tpu_sc as plsc

VOCAB = 1000000
EMBED_DIM = 32
BATCH = 4096
HIST = 200

_INFO = plsc.get_sparse_core_info()
NC = _INFO.num_cores
NS = _INFO.num_subcores
NW = NC * NS

B_TOTAL = BATCH * HIST
ROWS_PER_GATHER = 128
H_TILES = HIST // 8
B_BLOCKS = BATCH // ROWS_PER_GATHER
CHUNK_GATHERS = 10
CHUNK_ROWS = CHUNK_GATHERS * ROWS_PER_GATHER
N_CHUNKS = HIST // CHUNK_GATHERS


def _make_gather():
    mesh = plsc.VectorSubcoreMesh(core_axis_name="c", subcore_axis_name="s")

    @functools.partial(
        pl.kernel,
        out_type=jax.ShapeDtypeStruct((B_TOTAL, EMBED_DIM), jnp.float32),
        mesh=mesh,
        scratch_types=[
            pltpu.VMEM((H_TILES, 8, ROWS_PER_GATHER), jnp.int32),
            pltpu.VMEM((CHUNK_ROWS, EMBED_DIM), jnp.float32),
            pltpu.SemaphoreType.DMA,
        ],
        compiler_params=pltpu.CompilerParams(use_tc_tiling_on_sc=False),
    )
    def k(table_hbm, idx_hbm, out_hbm, idx_v, rows_v, sem):
        wid = lax.axis_index("s") * NC + lax.axis_index("c")
        pltpu.sync_copy(idx_hbm.at[:, :, wid], idx_v)
        col_base = wid * ROWS_PER_GATHER

        def chunk_body(c, _):
            descs = []
            for g in range(CHUNK_GATHERS):
                h = c * CHUNK_GATHERS + g
                descs.append(pltpu.async_copy(
                    table_hbm.at[idx_v.at[h // 8, h % 8]],
                    rows_v.at[pl.ds(g * ROWS_PER_GATHER, ROWS_PER_GATHER)],
                    sem))
            for d in descs:
                d.wait()
            for g in range(CHUNK_GATHERS):
                h = c * CHUNK_GATHERS + g
                pltpu.sync_copy(
                    rows_v.at[pl.ds(g * ROWS_PER_GATHER, ROWS_PER_GATHER)],
                    out_hbm.at[pl.ds(h * BATCH + col_base, ROWS_PER_GATHER)])
            return ()

        lax.fori_loop(0, N_CHUNKS, chunk_body, ())

    return k


_gather = _make_gather()


def kernel(inp, lengths, table):
    del lengths
    idx = jnp.clip(inp.astype(jnp.int32), 0, VOCAB - 1)
    idx = idx.T.reshape(H_TILES, 8, B_BLOCKS, ROWS_PER_GATHER)
    out = _gather(table, idx)
    return out.reshape(HIST, BATCH, EMBED_DIM).transpose(1, 0, 2)


# speedup vs baseline: 1.0001x; 1.0001x over previous
import functools

import jax
import jax.numpy as jnp
from jax import lax
from jax.experimental import pallas as pl
from jax.experimental.pallas import tpu as pltpu
from jax.experimental.pallas import tpu_sc as plsc

VOCAB = 1000000
EMBED_DIM = 32
BATCH = 4096
HIST = 200

_INFO = plsc.get_sparse_core_info()
NC = _INFO.num_cores
NS = _INFO.num_subcores
NW = NC * NS

B_TOTAL = BATCH * HIST
ROWS_PER_GATHER = 128
H_TILES = HIST // 8
B_BLOCKS = BATCH // ROWS_PER_GATHER
CHUNK_GATHERS = 10
CHUNK_ROWS = CHUNK_GATHERS * ROWS_PER_GATHER
N_CHUNKS = HIST // CHUNK_GATHERS


def _make_gather():
    mesh = plsc.VectorSubcoreMesh(core_axis_name="c", subcore_axis_name="s")

    @functools.partial(
        pl.kernel,
        out_type=jax.ShapeDtypeStruct((B_TOTAL, EMBED_DIM), jnp.float32),
        mesh=mesh,
        scratch_types=[
            pltpu.VMEM((H_TILES, 8, ROWS_PER_GATHER), jnp.int32),
            pltpu.VMEM((CHUNK_ROWS, EMBED_DIM), jnp.float32),
            pltpu.SemaphoreType.DMA,
        ],
        compiler_params=pltpu.CompilerParams(use_tc_tiling_on_sc=False),
    )
    def k(table_hbm, idx_hbm, out_hbm, idx_v, rows_v, sem):
        wid = lax.axis_index("s") * NC + lax.axis_index("c")
        pltpu.sync_copy(idx_hbm.at[:, wid], idx_v)
        col_base = wid * ROWS_PER_GATHER

        def chunk_body(c, _):
            descs = []
            for g in range(CHUNK_GATHERS):
                h = c * CHUNK_GATHERS + g
                descs.append(pltpu.async_copy(
                    table_hbm.at[idx_v.at[h // 8, h % 8]],
                    rows_v.at[pl.ds(g * ROWS_PER_GATHER, ROWS_PER_GATHER)],
                    sem))
            for d in descs:
                d.wait()
            for g in range(CHUNK_GATHERS):
                h = c * CHUNK_GATHERS + g
                pltpu.sync_copy(
                    rows_v.at[pl.ds(g * ROWS_PER_GATHER, ROWS_PER_GATHER)],
                    out_hbm.at[pl.ds(h * BATCH + col_base, ROWS_PER_GATHER)])
            return ()

        lax.fori_loop(0, N_CHUNKS, chunk_body, ())

    return k


_gather = _make_gather()


def kernel(inp, lengths, table):
    del lengths
    idx = jnp.clip(inp.astype(jnp.int32), 0, VOCAB - 1)
    idx = (idx.T.reshape(H_TILES, 8, B_BLOCKS, ROWS_PER_GATHER)
           .transpose(0, 2, 1, 3))
    out = _gather(table, idx)
    return out.reshape(HIST, BATCH, EMBED_DIM).transpose(1, 0, 2)


# speedup vs baseline: 1.0002x; 1.0001x over previous
import functools

import jax
import jax.numpy as jnp
from jax import lax
from jax.experimental import pallas as pl
from jax.experimental.pallas import tpu as pltpu
from jax.experimental.pallas import tpu_sc as plsc

VOCAB = 1000000
EMBED_DIM = 32
BATCH = 4096
HIST = 200

_INFO = plsc.get_sparse_core_info()
NC = _INFO.num_cores
NS = _INFO.num_subcores
NW = NC * NS

B_TOTAL = BATCH * HIST
ROWS_PER_GATHER = 128
H_TILES = HIST // 8
B_BLOCKS = BATCH // ROWS_PER_GATHER
CHUNK_GATHERS = 10
CHUNK_ROWS = CHUNK_GATHERS * ROWS_PER_GATHER
N_CHUNKS = HIST // CHUNK_GATHERS


def _make_gather():
    mesh = plsc.VectorSubcoreMesh(core_axis_name="c", subcore_axis_name="s")

    @functools.partial(
        pl.kernel,
        out_type=jax.ShapeDtypeStruct((HIST, BATCH, EMBED_DIM), jnp.float32),
        mesh=mesh,
        scratch_types=[
            pltpu.VMEM((H_TILES, 8, ROWS_PER_GATHER), jnp.int32),
            pltpu.VMEM((CHUNK_ROWS, EMBED_DIM), jnp.float32),
            pltpu.SemaphoreType.DMA,
        ],
        compiler_params=pltpu.CompilerParams(use_tc_tiling_on_sc=False),
    )
    def k(table_hbm, idx_hbm, out_hbm, idx_v, rows_v, sem):
        wid = lax.axis_index("s") * NC + lax.axis_index("c")
        pltpu.sync_copy(idx_hbm.at[:, wid], idx_v)
        col_base = wid * ROWS_PER_GATHER

        def chunk_body(c, _):
            descs = []
            for g in range(CHUNK_GATHERS):
                h = c * CHUNK_GATHERS + g
                descs.append(pltpu.async_copy(
                    table_hbm.at[idx_v.at[h // 8, h % 8]],
                    rows_v.at[pl.ds(g * ROWS_PER_GATHER, ROWS_PER_GATHER)],
                    sem))
            for d in descs:
                d.wait()
            for g in range(CHUNK_GATHERS):
                h = c * CHUNK_GATHERS + g
                pltpu.sync_copy(
                    rows_v.at[pl.ds(g * ROWS_PER_GATHER, ROWS_PER_GATHER)],
                    out_hbm.at[h, pl.ds(col_base, ROWS_PER_GATHER)])
            return ()

        lax.fori_loop(0, N_CHUNKS, chunk_body, ())

    return k


_gather = _make_gather()


def kernel(inp, lengths, table):
    del lengths
    idx = jnp.clip(inp.astype(jnp.int32), 0, VOCAB - 1)
    idx = (idx.T.reshape(H_TILES, 8, B_BLOCKS, ROWS_PER_GATHER)
           .transpose(0, 2, 1, 3))
    out = _gather(table, idx)
    return out.transpose(1, 0, 2)
